# baseline (speedup 1.0000x reference)
import jax
import jax.numpy as jnp
from jax import lax
from jax.experimental import pallas as pl
from jax.experimental.pallas import tpu as pltpu
from jax.experimental.pallas import tpu_sc as plsc

B = 16384
V = 1000000
D = 32
W = 128
NC = 2
NS = 16
NW = NC * NS
BW_BLK = 512
SH_BLK = 9
NBUF = 4
NBLK = V // BW_BLK
TAIL0 = NBLK * BW_BLK
BPW = B // NW
BLK_PW = (NBLK + NW - 1) // NW
NVREG = B // 16
WAVE = 128
FLUSH_AT = WAVE - 16

_params = pltpu.CompilerParams(use_tc_tiling_on_sc=True, needs_layout_passes=False)


def _pass1(tt, uidx, inter, idx_v, hitb_v, blk0, blk1, blk2, blk3, hrow, hidx,
           sm0, sm1, sm2, sm3, ssem):
    c = lax.axis_index("c")
    s = lax.axis_index("s")
    w = s * NC + c
    k0 = w * BLK_PW
    k1 = jnp.minimum(k0 + BLK_PW, NBLK)
    nblk = k1 - k0

    bufs = [blk0, blk1, blk2, blk3]
    sems = [sm0, sm1, sm2, sm3]

    pltpu.sync_copy(uidx, idx_v)

    lanes = lax.iota(jnp.int32, 16)

    def scan_body(g, nh):
        u = idx_v[pl.ds(g * 16, 16)]
        kh = lax.shift_right_logical(u, SH_BLK)
        m = (kh >= k0) & (kh < k1)
        cnt = plsc.all_reduce_population_count(m)
        plsc.store_compressed(hitb_v.at[pl.ds(nh, 16)], g * 16 + lanes, mask=m)
        return nh + jnp.max(cnt)

    nh = lax.fori_loop(0, NVREG, scan_body, jnp.int32(0))
    nhv = (nh + 15) // 16

    def fire(kk, j):
        col = pl.multiple_of(kk * BW_BLK, BW_BLK)
        return pltpu.async_copy(tt.at[:, pl.ds(col, BW_BLK)], bufs[j], sems[j])

    def wait_slot(kk, j):
        col = pl.multiple_of(kk * BW_BLK, BW_BLK)
        pltpu.make_async_copy(
            tt.at[:, pl.ds(col, BW_BLK)], bufs[j], sems[j]).wait()

    def do_flush(wpf):
        def pad_body(q, _):
            fill = jnp.full((16,), B, jnp.int32) + q * 16 + lanes
            cur = hidx[pl.ds(q * 16, 16)]
            sel = (q * 16 + lanes) >= wpf
            hidx[pl.ds(q * 16, 16)] = jnp.where(sel, fill, cur)
            return _

        lax.fori_loop(0, WAVE // 16, pad_body, 0)
        pltpu.async_copy(hrow, inter.at[hidx.at[pl.ds(0, WAVE)]], ssem).wait()

    for j in range(NBUF - 1):
        @pl.when(j < nblk)
        def _(j=j):
            fire(k0 + j, j)

    def blk_body(i, wp):
        k = k0 + i
        slot = lax.rem(i, NBUF)
        for j in range(NBUF):
            @pl.when(slot == j)
            def _(j=j):
                wait_slot(k, j)

        nxt = lax.rem(i + NBUF - 1, NBUF)

        @pl.when(i + NBUF - 1 < nblk)
        def _():
            for j in range(NBUF):
                @pl.when(nxt == j)
                def _(j=j):
                    fire(k + NBUF - 1, j)

        def hv_body(hv, wp2):
            bpos = hitb_v[pl.ds(hv * 16, 16)]
            valid = (hv * 16 + lanes) < nh
            u = plsc.load_gather(idx_v, [bpos], mask=valid)
            m = valid & (lax.shift_right_logical(u, SH_BLK) == k)

            def match_body(state):
                m2, wp3 = state
                pv = plsc.all_reduce_ffs(m2)
                p = jnp.max(pv)
                sel1 = lanes == p
                u_s = jnp.max(jnp.where(sel1, u, jnp.int32(-1)))
                c_s = lax.rem(u_s, BW_BLK)
                cs = jnp.full((16,), c_s, jnp.int32)

                def extract(bref):
                    v0 = plsc.load_gather(bref, [lanes, cs])
                    v1 = plsc.load_gather(bref, [16 + lanes, cs])
                    hrow[wp3, pl.ds(0, 16)] = v0
                    hrow[wp3, pl.ds(16, 16)] = v1

                for j in range(NBUF):
                    @pl.when(slot == j)
                    def _(j=j):
                        extract(bufs[j])

                plsc.store_compressed(hidx.at[pl.ds(wp3, 16)], bpos, mask=sel1)
                return m2 & jnp.logical_not(sel1), wp3 + 1

            m_fin, wp4 = lax.while_loop(
                lambda st: jnp.any(st[0]), match_body, (m, wp2))

            @pl.when(wp4 >= FLUSH_AT)
            def _():
                do_flush(wp4)

            return jnp.where(wp4 >= FLUSH_AT, jnp.int32(0), wp4)

        return lax.fori_loop(0, nhv, hv_body, wp)

    wp_f = lax.fori_loop(0, nblk, blk_body, jnp.int32(0))

    do_flush(wp_f)


def _pass2(inter, uidx, gidx, aidx, tail, tg, ta, ot,
           st_v, ui_v, gi_v, ai_v, tail_v, tg_v, ta_v, otv, sem):
    c = lax.axis_index("c")
    s = lax.axis_index("s")
    w = s * NC + c
    base = w * BPW

    pltpu.sync_copy(uidx.at[pl.ds(base, BPW)], ui_v)
    pltpu.sync_copy(gidx.at[pl.ds(base, BPW)], gi_v)
    pltpu.sync_copy(aidx.at[pl.ds(base, BPW)], ai_v)
    pltpu.sync_copy(tail, tail_v)
    pltpu.sync_copy(tg, tg_v)
    pltpu.sync_copy(ta, ta_v)

    lanes = lax.iota(jnp.int32, 16)
    HALF = 256

    for h in range(BPW // HALF):
        pltpu.sync_copy(inter.at[pl.ds(base + h * HALF, HALF)], st_v)

        def grp_body(g, carry):
            rows = g * 16 + lanes
            u = ui_v[pl.ds(h * HALF + g * 16, 16)]
            gi = gi_v[pl.ds(h * HALF + g * 16, 16)]
            ai = ai_v[pl.ds(h * HALF + g * 16, 16)]
            mt = u >= TAIL0
            tcl = jnp.clip(u - TAIL0, 0, V - TAIL0 - 1)
            for f in range(D):
                fs = jnp.full((16,), f, jnp.int32)
                v = plsc.load_gather(st_v, [rows, fs])
                vt = plsc.load_gather(tail_v, [tcl, fs])
                otv[f, pl.ds(g * 16, 16)] = jnp.where(mt, vt, v)
                vg = plsc.load_gather(tg_v, [gi, fs])
                otv[D + f, pl.ds(g * 16, 16)] = vg
                va = plsc.load_gather(ta_v, [ai, fs])
                otv[2 * D + f, pl.ds(g * 16, 16)] = va
            return carry

        lax.fori_loop(0, HALF // 16, grp_body, 0)
        pltpu.sync_copy(otv, ot.at[:, pl.ds(base + h * HALF, HALF)])


def kernel(user_idx, gender_idx, age_idx, table_users, table_gender, table_age):
    mesh = plsc.VectorSubcoreMesh(core_axis_name="c", subcore_axis_name="s")
    tt = table_users.T
    tail = table_users[TAIL0:]

    f1 = pl.kernel(
        _pass1,
        mesh=mesh,
        compiler_params=_params,
        out_type=jax.ShapeDtypeStruct((B + WAVE, W), jnp.float32),
        scratch_types=[
            pltpu.VMEM((B,), jnp.int32),
            pltpu.VMEM((B,), jnp.int32),
            pltpu.VMEM((D, BW_BLK), jnp.float32),
            pltpu.VMEM((D, BW_BLK), jnp.float32),
            pltpu.VMEM((D, BW_BLK), jnp.float32),
            pltpu.VMEM((D, BW_BLK), jnp.float32),
            pltpu.VMEM((WAVE, W), jnp.float32),
            pltpu.VMEM((WAVE + 16,), jnp.int32),
            pltpu.SemaphoreType.DMA,
            pltpu.SemaphoreType.DMA,
            pltpu.SemaphoreType.DMA,
            pltpu.SemaphoreType.DMA,
            pltpu.SemaphoreType.DMA,
        ],
    )
    inter = f1(tt, user_idx)

    f2 = pl.kernel(
        _pass2,
        mesh=mesh,
        compiler_params=_params,
        out_type=jax.ShapeDtypeStruct((3 * D, B), jnp.float32),
        scratch_types=[
            pltpu.VMEM((256, W), jnp.float32),
            pltpu.VMEM((BPW,), jnp.int32),
            pltpu.VMEM((BPW,), jnp.int32),
            pltpu.VMEM((BPW,), jnp.int32),
            pltpu.VMEM((V - TAIL0, D), jnp.float32),
            pltpu.VMEM((2, D), jnp.float32),
            pltpu.VMEM((7, D), jnp.float32),
            pltpu.VMEM((3 * D, 256), jnp.float32),
            pltpu.SemaphoreType.DMA,
        ],
    )
    ot = f2(inter, user_idx, gender_idx, age_idx, tail, table_gender, table_age)
    return ot.T


# speedup vs baseline: 1.1533x; 1.1533x over previous
import jax
import jax.numpy as jnp
from jax import lax
from jax.experimental import pallas as pl
from jax.experimental.pallas import tpu as pltpu
from jax.experimental.pallas import tpu_sc as plsc

B = 16384
V = 1000000
D = 32
W = 128
NC = 2
NS = 16
NW = NC * NS
BW_BLK = 512
SH_BLK = 9
NBUF = 4
NBLK = V // BW_BLK
TAIL0 = NBLK * BW_BLK
BPW = B // NW
BLK_PW = (NBLK + NW - 1) // NW
NVREG = B // 16
WAVE = 128
FLUSH_AT = WAVE - 16

_params = pltpu.CompilerParams(use_tc_tiling_on_sc=True, needs_layout_passes=False)


def _pass1(tt, uidx, inter, idx_v, hitu_v, hitb_v, blk0, blk1, hrow, hidx,
           sm0, sm1, ssem):
    c = lax.axis_index("c")
    s = lax.axis_index("s")
    w = s * NC + c
    k0 = w * BLK_PW
    k1 = jnp.minimum(k0 + BLK_PW, NBLK)
    nblk = k1 - k0

    bufs = [blk0, blk1]
    sems = [sm0, sm1]

    pltpu.sync_copy(uidx, idx_v)

    lanes = lax.iota(jnp.int32, 16)

    def scan_body(g, nh):
        u = idx_v[pl.ds(g * 16, 16)]
        kh = lax.shift_right_logical(u, SH_BLK)
        m = (kh >= k0) & (kh < k1)
        cnt = plsc.all_reduce_population_count(m)
        plsc.store_compressed(hitu_v.at[pl.ds(nh, 16)], u, mask=m)
        plsc.store_compressed(hitb_v.at[pl.ds(nh, 16)], g * 16 + lanes, mask=m)
        return nh + jnp.max(cnt)

    nh = lax.fori_loop(0, NVREG, scan_body, jnp.int32(0))
    nhv = (nh + 15) // 16

    def fire(kk, j):
        col = pl.multiple_of(kk * BW_BLK, BW_BLK)
        return pltpu.async_copy(tt.at[:, pl.ds(col, BW_BLK)], bufs[j], sems[j])

    def wait_slot(kk, j):
        col = pl.multiple_of(kk * BW_BLK, BW_BLK)
        pltpu.make_async_copy(
            tt.at[:, pl.ds(col, BW_BLK)], bufs[j], sems[j]).wait()

    def do_flush(wpf):
        def pad_body(q, _):
            fill = jnp.full((16,), B, jnp.int32) + q * 16 + lanes
            cur = hidx[pl.ds(q * 16, 16)]
            sel = (q * 16 + lanes) >= wpf
            hidx[pl.ds(q * 16, 16)] = jnp.where(sel, fill, cur)
            return _

        lax.fori_loop(0, WAVE // 16, pad_body, 0)
        pltpu.async_copy(hrow, inter.at[hidx.at[pl.ds(0, WAVE)]], ssem).wait()

    fire(k0, 0)

    def blk_body(i, wp):
        k = k0 + i
        slot = lax.rem(i, 2)
        for j in range(2):
            @pl.when(slot == j)
            def _(j=j):
                wait_slot(k, j)

        @pl.when((i + 1 < nblk) & (slot == 0))
        def _():
            fire(k + 1, 1)

        @pl.when((i + 1 < nblk) & (slot == 1))
        def _():
            fire(k + 1, 0)

        def hv_body(hv, wp2):
            u = hitu_v[pl.ds(hv * 16, 16)]
            bpos = hitb_v[pl.ds(hv * 16, 16)]
            valid = (hv * 16 + lanes) < nh
            m = valid & (lax.shift_right_logical(u, SH_BLK) == k)

            def match_body(state):
                m2, wp3 = state
                pv = plsc.all_reduce_ffs(m2)
                p = jnp.max(pv)
                sel1 = lanes == p
                u_s = jnp.max(jnp.where(sel1, u, jnp.int32(-1)))
                c_s = lax.rem(u_s, BW_BLK)
                cs = jnp.full((16,), c_s, jnp.int32)

                def extract(bref):
                    v0 = plsc.load_gather(bref, [lanes, cs])
                    v1 = plsc.load_gather(bref, [16 + lanes, cs])
                    hrow[wp3, pl.ds(0, 16)] = v0
                    hrow[wp3, pl.ds(16, 16)] = v1

                for j in range(2):
                    @pl.when(slot == j)
                    def _(j=j):
                        extract(bufs[j])

                plsc.store_compressed(hidx.at[pl.ds(wp3, 16)], bpos, mask=sel1)
                return m2 & jnp.logical_not(sel1), wp3 + 1

            m_fin, wp4 = lax.while_loop(
                lambda st: jnp.any(st[0]), match_body, (m, wp2))

            @pl.when(wp4 >= FLUSH_AT)
            def _():
                do_flush(wp4)

            return jnp.where(wp4 >= FLUSH_AT, jnp.int32(0), wp4)

        return lax.fori_loop(0, nhv, hv_body, wp)

    wp_f = lax.fori_loop(0, nblk, blk_body, jnp.int32(0))

    do_flush(wp_f)


def _pass2(inter, uidx, gidx, aidx, tail, tg, ta, ot,
           st0, st1, ui_v, gi_v, ai_v, tail_v, tg_v, ta_v, otv0, otv1,
           sin, si0, si1, so0, so1):
    c = lax.axis_index("c")
    s = lax.axis_index("s")
    w = s * NC + c
    base = w * BPW

    ins = [
        (uidx.at[pl.ds(base, BPW)], ui_v),
        (gidx.at[pl.ds(base, BPW)], gi_v),
        (aidx.at[pl.ds(base, BPW)], ai_v),
        (tail, tail_v),
        (tg, tg_v),
        (ta, ta_v),
    ]
    for src, dst in ins:
        pltpu.async_copy(src, dst, sin)

    lanes = lax.iota(jnp.int32, 16)
    HALF = 256
    NH2 = BPW // HALF
    sts = [st0, st1]
    otvs = [otv0, otv1]
    sis = [si0, si1]
    sos = [so0, so1]

    pltpu.async_copy(inter.at[pl.ds(base, HALF)], st0, si0)
    for src, dst in ins:
        pltpu.make_async_copy(src, dst, sin).wait()

    for h in range(NH2):
        cur, nxt = h % 2, (h + 1) % 2
        st_v = sts[cur]
        otv = otvs[cur]
        pltpu.make_async_copy(
            inter.at[pl.ds(base + h * HALF, HALF)], st_v, sis[cur]).wait()
        if h + 1 < NH2:
            pltpu.async_copy(
                inter.at[pl.ds(base + (h + 1) * HALF, HALF)], sts[nxt], sis[nxt])

        def grp_body(g, carry):
            rows = g * 16 + lanes
            u = ui_v[pl.ds(h * HALF + g * 16, 16)]
            gi = gi_v[pl.ds(h * HALF + g * 16, 16)]
            ai = ai_v[pl.ds(h * HALF + g * 16, 16)]
            mt = u >= TAIL0
            tcl = jnp.clip(u - TAIL0, 0, V - TAIL0 - 1)
            for f in range(D):
                fs = jnp.full((16,), f, jnp.int32)
                v = plsc.load_gather(st_v, [rows, fs])
                vt = plsc.load_gather(tail_v, [tcl, fs])
                otv[f, pl.ds(g * 16, 16)] = jnp.where(mt, vt, v)
                vg = plsc.load_gather(tg_v, [gi, fs])
                otv[D + f, pl.ds(g * 16, 16)] = vg
                va = plsc.load_gather(ta_v, [ai, fs])
                otv[2 * D + f, pl.ds(g * 16, 16)] = va
            return carry

        lax.fori_loop(0, HALF // 16, grp_body, 0)
        pltpu.async_copy(otv, ot.at[:, pl.ds(base + h * HALF, HALF)], sos[cur])

    for h in range(NH2):
        pltpu.make_async_copy(
            otvs[h % 2], ot.at[:, pl.ds(base + h * HALF, HALF)],
            sos[h % 2]).wait()


def kernel(user_idx, gender_idx, age_idx, table_users, table_gender, table_age):
    mesh = plsc.VectorSubcoreMesh(core_axis_name="c", subcore_axis_name="s")
    tt = table_users.T
    tail = table_users[TAIL0:]

    f1 = pl.kernel(
        _pass1,
        mesh=mesh,
        compiler_params=_params,
        out_type=jax.ShapeDtypeStruct((B + WAVE, W), jnp.float32),
        scratch_types=[
            pltpu.VMEM((B,), jnp.int32),
            pltpu.VMEM((B,), jnp.int32),
            pltpu.VMEM((B,), jnp.int32),
            pltpu.VMEM((D, BW_BLK), jnp.float32),
            pltpu.VMEM((D, BW_BLK), jnp.float32),
            pltpu.VMEM((WAVE, W), jnp.float32),
            pltpu.VMEM((WAVE + 16,), jnp.int32),
            pltpu.SemaphoreType.DMA,
            pltpu.SemaphoreType.DMA,
            pltpu.SemaphoreType.DMA,
        ],
    )
    inter = f1(tt, user_idx)

    f2 = pl.kernel(
        _pass2,
        mesh=mesh,
        compiler_params=_params,
        out_type=jax.ShapeDtypeStruct((3 * D, B), jnp.float32),
        scratch_types=[
            pltpu.VMEM((256, W), jnp.float32),
            pltpu.VMEM((256, W), jnp.float32),
            pltpu.VMEM((BPW,), jnp.int32),
            pltpu.VMEM((BPW,), jnp.int32),
            pltpu.VMEM((BPW,), jnp.int32),
            pltpu.VMEM((V - TAIL0, D), jnp.float32),
            pltpu.VMEM((2, D), jnp.float32),
            pltpu.VMEM((7, D), jnp.float32),
            pltpu.VMEM((3 * D, 256), jnp.float32),
            pltpu.VMEM((3 * D, 256), jnp.float32),
            pltpu.SemaphoreType.DMA,
            pltpu.SemaphoreType.DMA,
            pltpu.SemaphoreType.DMA,
            pltpu.SemaphoreType.DMA,
            pltpu.SemaphoreType.DMA,
        ],
    )
    ot = f2(inter, user_idx, gender_idx, age_idx, tail, table_gender, table_age)
    return ot.T
